# baseline (speedup 1.0000x reference)
import functools

import jax
import jax.numpy as jnp
from jax import lax
from jax.experimental import pallas as pl
from jax.experimental.pallas import tpu as pltpu
from jax.experimental.pallas import tpu_sc as plsc

K = 2048
L = 16
NW = 32
BLK = 2048
UNROLL = 4


def _tec_body(N, D, rows_per_worker,
              mask_hbm, rule_hbm, body_hbm,
              body_out_hbm, mask_out_hbm, rule_out_hbm,
              mask_v, rule_v, comb_v, sel_v, mko_v, rlo_v,
              obt0_v, obt1_v,
              msem0, msem1, rsem, psem0, psem1, wsem0, wsem1, osem):
    nblocks = N // BLK
    wid = lax.axis_index("s") * 2 + lax.axis_index("c")
    iota = lax.iota(jnp.int32, L)
    msems = (msem0, msem1)
    psems = (psem0, psem1)
    wsems = (wsem0, wsem1)
    obts = (obt0_v, obt1_v)

    def when(pred, fn):
        lax.cond(pred, fn, lambda: None)

    def do_row(row):
        def m_copy(b, sem):
            return pltpu.make_async_copy(
                mask_hbm.at[row, pl.ds(b * BLK, BLK)],
                mask_v.at[pl.ds(b * BLK, BLK)], sem)

        def r_copy(b):
            return pltpu.make_async_copy(
                rule_hbm.at[row, pl.ds(b * BLK, BLK)],
                rule_v.at[pl.ds(b * BLK, BLK)], rsem)

        m_copy(0, msems[0]).start()

        def scan_chunk(base_elem, n1, n0):
            m = mask_v[pl.ds(base_elem, L)]
            incl = plsc.cumsum(m)
            excl = incl - m
            s = jnp.sum(m)
            g = base_elem + iota
            pos1 = n1 + excl
            pos0 = n0 + (iota - excl)
            m1 = (m > 0) & (pos1 < K)
            m0 = (m == 0) & (pos0 < K)
            plsc.store_scatter(comb_v, [pos1], g, mask=m1)
            plsc.store_scatter(comb_v, [pos0 + K], g, mask=m0)
            return n1 + s, n0 + (L - s)

        carry = (0, 0, 0)
        for b in range(nblocks):
            n1, n0, nb = carry

            def live(c, b=b):
                n1, n0 = c
                m_copy(b, msems[b % 2]).wait()
                if b + 1 < nblocks:
                    m_copy(b + 1, msems[(b + 1) % 2]).start()
                r_copy(b).start()

                def step(i, c):
                    n1, n0 = c
                    base = b * BLK + i * (UNROLL * L)
                    for u in range(UNROLL):
                        n1, n0 = scan_chunk(base + u * L, n1, n0)
                    return n1, n0

                n1, n0 = lax.fori_loop(0, BLK // (UNROLL * L), step, (n1, n0))
                return n1, n0, b + 1

            carry = lax.cond(
                (n1 < K) | (n0 < K), live, lambda c: (c[0], c[1], nb), (n1, n0)
            )
        n1f, _, nb = carry
        n1e = jnp.minimum(n1f, K)

        for b in range(1, nblocks):
            when(nb == b, lambda b=b: m_copy(b, msems[b % 2]).wait())

        def drain_rule(b, _):
            when(b < nb, lambda: r_copy(b).wait())
            return 0

        lax.fori_loop(0, nblocks, drain_rule, 0)

        def combine(t, _):
            j = t * L + iota
            take1 = j < n1e
            src = jnp.where(take1, j, j - n1e + K)
            sel = plsc.load_gather(comb_v, [src])
            sel_v[pl.ds(t * L, L)] = sel
            rlo_v[pl.ds(t * L, L)] = plsc.load_gather(rule_v, [sel])
            mko_v[pl.ds(t * L, L)] = take1.astype(jnp.int32)
            return 0

        lax.fori_loop(0, K // L, combine, 0)
        pltpu.make_async_copy(mko_v, mask_out_hbm.at[row], osem).start()
        pltpu.make_async_copy(rlo_v, rule_out_hbm.at[row], osem).start()

        last1 = plsc.load_gather(
            comb_v, [jnp.broadcast_to(jnp.maximum(n1e - 1, 0), (L,))])
        last0 = plsc.load_gather(
            comb_v, [jnp.broadcast_to(
                jnp.clip(2 * K - n1e - 1, K, 2 * K - 1), (L,))])
        pe1 = jnp.where(n1e > 0, jnp.max(last1), 0)
        pe0 = jnp.where(n1e < K, jnp.max(last0), 0)
        pex = jnp.maximum(pe1, pe0) + 1
        nbb = (pex + BLK - 1) // BLK

        def p_copy(d, b):
            buf = rule_v if d % 2 == 0 else mask_v
            return pltpu.make_async_copy(
                body_hbm.at[row, d, pl.ds(b * BLK, BLK)],
                buf.at[pl.ds(b * BLK, BLK)], psems[d % 2])

        def fire_plane(d):
            def fb(b, _):
                when(b < nbb, lambda: p_copy(d, b).start())
                return 0

            lax.fori_loop(0, nblocks, fb, 0)

        def wait_plane(d):
            def wb(b, _):
                when(b < nbb, lambda: p_copy(d, b).wait())
                return 0

            lax.fori_loop(0, nblocks, wb, 0)

        fire_plane(0)
        for d in range(D):
            wait_plane(d)
            if d + 1 < D:
                fire_plane(d + 1)
            if d >= 2:
                pltpu.make_async_copy(
                    obts[d % 2], body_out_hbm.at[row, d - 2], wsems[d % 2]
                ).wait()
            obt = obts[d % 2]
            pbuf = rule_v if d % 2 == 0 else mask_v

            def extract(t, _, obt=obt, pbuf=pbuf):
                nvec = sel_v[pl.ds(t * L, L)]
                obt[pl.ds(t * L, L)] = plsc.load_gather(pbuf, [nvec])
                return 0

            lax.fori_loop(0, K // L, extract, 0)
            pltpu.make_async_copy(
                obt, body_out_hbm.at[row, d], wsems[d % 2]).start()

        for d in (D - 2, D - 1):
            pltpu.make_async_copy(
                obts[d % 2], body_out_hbm.at[row, d], wsems[d % 2]).wait()
        pltpu.make_async_copy(mko_v, mask_out_hbm.at[row], osem).wait()
        pltpu.make_async_copy(rlo_v, rule_out_hbm.at[row], osem).wait()

    def row_iter(r, _):
        do_row(wid * rows_per_worker + r)
        return 0

    lax.fori_loop(0, rows_per_worker, row_iter, 0)


@jax.jit
def kernel(body, mask, rule_idx):
    B, N, D = body.shape
    rows_per_worker = B // NW
    rdt = rule_idx.dtype
    mask_i = mask.astype(jnp.int32)
    rule_i = rule_idx.astype(jnp.int32)
    body_t = lax.bitcast_convert_type(body.transpose(0, 2, 1), jnp.int32)

    mesh = plsc.VectorSubcoreMesh(
        core_axis_name="c", subcore_axis_name="s", num_cores=2, num_subcores=16
    )
    body_o, mask_o, rule_o = pl.kernel(
        functools.partial(_tec_body, N, D, rows_per_worker),
        out_type=(
            jax.ShapeDtypeStruct((B, D, K), jnp.int32),
            jax.ShapeDtypeStruct((B, K), jnp.int32),
            jax.ShapeDtypeStruct((B, K), jnp.int32),
        ),
        mesh=mesh,
        compiler_params=pltpu.CompilerParams(
            needs_layout_passes=False, use_tc_tiling_on_sc=True
        ),
        scratch_types=[
            pltpu.VMEM((N,), jnp.int32),
            pltpu.VMEM((N,), jnp.int32),
            pltpu.VMEM((2 * K,), jnp.int32),
            pltpu.VMEM((K,), jnp.int32),
            pltpu.VMEM((K,), jnp.int32),
            pltpu.VMEM((K,), jnp.int32),
            pltpu.VMEM((K,), jnp.int32),
            pltpu.VMEM((K,), jnp.int32),
            pltpu.SemaphoreType.DMA,
            pltpu.SemaphoreType.DMA,
            pltpu.SemaphoreType.DMA,
            pltpu.SemaphoreType.DMA,
            pltpu.SemaphoreType.DMA,
            pltpu.SemaphoreType.DMA,
            pltpu.SemaphoreType.DMA,
            pltpu.SemaphoreType.DMA,
        ],
    )(mask_i, rule_i, body_t)
    body_f = lax.bitcast_convert_type(body_o, jnp.float32).transpose(0, 2, 1)
    return body_f, mask_o.astype(jnp.bool_), rule_o.astype(rdt)


# speedup vs baseline: 1.0027x; 1.0027x over previous
import functools

import jax
import jax.numpy as jnp
from jax import lax
from jax.experimental import pallas as pl
from jax.experimental.pallas import tpu as pltpu
from jax.experimental.pallas import tpu_sc as plsc

K = 2048
L = 16
NW = 32
BLK = 8192
UNROLL = 4


def _tec_body(N, D, rows_per_worker,
              mask_hbm, rule_hbm, body_hbm,
              body_out_hbm, mask_out_hbm, rule_out_hbm,
              mask_v, rule_v, comb_v, sel_v, mko_v, rlo_v,
              obt0_v, obt1_v,
              msem0, msem1, rsem, psem0, psem1, wsem0, wsem1, osem):
    nblocks = N // BLK
    wid = lax.axis_index("s") * 2 + lax.axis_index("c")
    iota = lax.iota(jnp.int32, L)
    msems = (msem0, msem1)
    psems = (psem0, psem1)
    wsems = (wsem0, wsem1)
    obts = (obt0_v, obt1_v)

    def when(pred, fn):
        lax.cond(pred, fn, lambda: None)

    def do_row(row):
        def m_copy(b, sem):
            return pltpu.make_async_copy(
                mask_hbm.at[row, pl.ds(b * BLK, BLK)],
                mask_v.at[pl.ds(b * BLK, BLK)], sem)

        def r_copy(b):
            return pltpu.make_async_copy(
                rule_hbm.at[row, pl.ds(b * BLK, BLK)],
                rule_v.at[pl.ds(b * BLK, BLK)], rsem)

        m_copy(0, msems[0]).start()

        def scan_chunk(base_elem, n1, n0):
            m = mask_v[pl.ds(base_elem, L)]
            incl = plsc.cumsum(m)
            excl = incl - m
            s = jnp.sum(m)
            g = base_elem + iota
            pos1 = n1 + excl
            pos0 = n0 + (iota - excl)
            m1 = (m > 0) & (pos1 < K)
            m0 = (m == 0) & (pos0 < K)
            plsc.store_scatter(comb_v, [pos1], g, mask=m1)
            plsc.store_scatter(comb_v, [pos0 + K], g, mask=m0)
            return n1 + s, n0 + (L - s)

        carry = (0, 0, 0)
        for b in range(nblocks):
            n1, n0, nb = carry

            def live(c, b=b):
                n1, n0 = c
                m_copy(b, msems[b % 2]).wait()
                if b + 1 < nblocks:
                    m_copy(b + 1, msems[(b + 1) % 2]).start()
                r_copy(b).start()

                def step(i, c):
                    n1, n0 = c
                    base = b * BLK + i * (UNROLL * L)
                    for u in range(UNROLL):
                        n1, n0 = scan_chunk(base + u * L, n1, n0)
                    return n1, n0

                n1, n0 = lax.fori_loop(0, BLK // (UNROLL * L), step, (n1, n0))
                return n1, n0, b + 1

            carry = lax.cond(
                (n1 < K) | (n0 < K), live, lambda c: (c[0], c[1], nb), (n1, n0)
            )
        n1f, _, nb = carry
        n1e = jnp.minimum(n1f, K)

        for b in range(1, nblocks):
            when(nb == b, lambda b=b: m_copy(b, msems[b % 2]).wait())

        def drain_rule(b, _):
            when(b < nb, lambda: r_copy(b).wait())
            return 0

        lax.fori_loop(0, nblocks, drain_rule, 0)

        def combine(t, _):
            j = t * L + iota
            take1 = j < n1e
            src = jnp.where(take1, j, j - n1e + K)
            sel = plsc.load_gather(comb_v, [src])
            sel_v[pl.ds(t * L, L)] = sel
            rlo_v[pl.ds(t * L, L)] = plsc.load_gather(rule_v, [sel])
            mko_v[pl.ds(t * L, L)] = take1.astype(jnp.int32)
            return 0

        lax.fori_loop(0, K // L, combine, 0)
        pltpu.make_async_copy(mko_v, mask_out_hbm.at[row], osem).start()
        pltpu.make_async_copy(rlo_v, rule_out_hbm.at[row], osem).start()

        last1 = plsc.load_gather(
            comb_v, [jnp.broadcast_to(jnp.maximum(n1e - 1, 0), (L,))])
        last0 = plsc.load_gather(
            comb_v, [jnp.broadcast_to(
                jnp.clip(2 * K - n1e - 1, K, 2 * K - 1), (L,))])
        pe1 = jnp.where(n1e > 0, jnp.max(last1), 0)
        pe0 = jnp.where(n1e < K, jnp.max(last0), 0)
        pex = jnp.maximum(pe1, pe0) + 1
        nbb = (pex + BLK - 1) // BLK

        def p_copy(d, b):
            buf = rule_v if d % 2 == 0 else mask_v
            return pltpu.make_async_copy(
                body_hbm.at[row, d, pl.ds(b * BLK, BLK)],
                buf.at[pl.ds(b * BLK, BLK)], psems[d % 2])

        def fire_plane(d):
            def fb(b, _):
                when(b < nbb, lambda: p_copy(d, b).start())
                return 0

            lax.fori_loop(0, nblocks, fb, 0)

        def wait_plane(d):
            def wb(b, _):
                when(b < nbb, lambda: p_copy(d, b).wait())
                return 0

            lax.fori_loop(0, nblocks, wb, 0)

        fire_plane(0)
        for d in range(D):
            wait_plane(d)
            if d + 1 < D:
                fire_plane(d + 1)
            if d >= 2:
                pltpu.make_async_copy(
                    obts[d % 2], body_out_hbm.at[row, d - 2], wsems[d % 2]
                ).wait()
            obt = obts[d % 2]
            pbuf = rule_v if d % 2 == 0 else mask_v

            def extract(t, _, obt=obt, pbuf=pbuf):
                nvec = sel_v[pl.ds(t * L, L)]
                obt[pl.ds(t * L, L)] = plsc.load_gather(pbuf, [nvec])
                return 0

            lax.fori_loop(0, K // L, extract, 0)
            pltpu.make_async_copy(
                obt, body_out_hbm.at[row, d], wsems[d % 2]).start()

        for d in (D - 2, D - 1):
            pltpu.make_async_copy(
                obts[d % 2], body_out_hbm.at[row, d], wsems[d % 2]).wait()
        pltpu.make_async_copy(mko_v, mask_out_hbm.at[row], osem).wait()
        pltpu.make_async_copy(rlo_v, rule_out_hbm.at[row], osem).wait()

    def row_iter(r, _):
        do_row(wid * rows_per_worker + r)
        return 0

    lax.fori_loop(0, rows_per_worker, row_iter, 0)


@jax.jit
def kernel(body, mask, rule_idx):
    B, N, D = body.shape
    rows_per_worker = B // NW
    rdt = rule_idx.dtype
    mask_i = mask.astype(jnp.int32)
    rule_i = rule_idx.astype(jnp.int32)
    body_t = lax.bitcast_convert_type(body.transpose(0, 2, 1), jnp.int32)

    mesh = plsc.VectorSubcoreMesh(
        core_axis_name="c", subcore_axis_name="s", num_cores=2, num_subcores=16
    )
    body_o, mask_o, rule_o = pl.kernel(
        functools.partial(_tec_body, N, D, rows_per_worker),
        out_type=(
            jax.ShapeDtypeStruct((B, D, K), jnp.int32),
            jax.ShapeDtypeStruct((B, K), jnp.int32),
            jax.ShapeDtypeStruct((B, K), jnp.int32),
        ),
        mesh=mesh,
        compiler_params=pltpu.CompilerParams(
            needs_layout_passes=False, use_tc_tiling_on_sc=True
        ),
        scratch_types=[
            pltpu.VMEM((N,), jnp.int32),
            pltpu.VMEM((N,), jnp.int32),
            pltpu.VMEM((2 * K,), jnp.int32),
            pltpu.VMEM((K,), jnp.int32),
            pltpu.VMEM((K,), jnp.int32),
            pltpu.VMEM((K,), jnp.int32),
            pltpu.VMEM((K,), jnp.int32),
            pltpu.VMEM((K,), jnp.int32),
            pltpu.SemaphoreType.DMA,
            pltpu.SemaphoreType.DMA,
            pltpu.SemaphoreType.DMA,
            pltpu.SemaphoreType.DMA,
            pltpu.SemaphoreType.DMA,
            pltpu.SemaphoreType.DMA,
            pltpu.SemaphoreType.DMA,
            pltpu.SemaphoreType.DMA,
        ],
    )(mask_i, rule_i, body_t)
    body_f = lax.bitcast_convert_type(body_o, jnp.float32).transpose(0, 2, 1)
    return body_f, mask_o.astype(jnp.bool_), rule_o.astype(rdt)


# speedup vs baseline: 2.0870x; 2.0814x over previous
import functools

import jax
import jax.numpy as jnp
from jax import lax
from jax.experimental import pallas as pl
from jax.experimental.pallas import tpu as pltpu
from jax.experimental.pallas import tpu_sc as plsc

K = 2048
L = 16
NW = 32
BLK = 8192
UNROLL = 4


def _tec_body(N, D, rows_per_worker,
              mask_hbm, rule_hbm, body_hbm,
              body_out_hbm, mask_out_hbm, rule_out_hbm,
              mask_v, rule_v, comb_v, sel_v, mko_v, rlo_v,
              obt0_v, obt1_v,
              msem0, msem1, rsem, psem0, psem1, wsem0, wsem1, osem):
    nblocks = N // BLK
    wid = lax.axis_index("s") * 2 + lax.axis_index("c")
    iota = lax.iota(jnp.int32, L)
    msems = (msem0, msem1)
    psems = (psem0, psem1)
    wsems = (wsem0, wsem1)
    obts = (obt0_v, obt1_v)

    def when(pred, fn):
        lax.cond(pred, fn, lambda: None)

    def do_row(row):
        def m_copy(b, sem):
            return pltpu.make_async_copy(
                mask_hbm.at[row, pl.ds(b * BLK, BLK)],
                mask_v.at[pl.ds(b * BLK, BLK)], sem)

        def r_copy(b):
            return pltpu.make_async_copy(
                rule_hbm.at[row, pl.ds(b * BLK, BLK)],
                rule_v.at[pl.ds(b * BLK, BLK)], rsem)

        m_copy(0, msems[0]).start()

        def scan_chunk(base_elem, n1, n0):
            m = mask_v[pl.ds(base_elem, L)]
            incl = plsc.cumsum(m)
            excl = incl - m
            s = jnp.sum(m)
            g = base_elem + iota
            pos1 = n1 + excl
            pos0 = n0 + (iota - excl)
            m1 = (m > 0) & (pos1 < K)
            m0 = (m == 0) & (pos0 < K)
            plsc.store_scatter(comb_v, [pos1], g, mask=m1)
            plsc.store_scatter(comb_v, [pos0 + K], g, mask=m0)
            return n1 + s, n0 + (L - s)

        carry = (0, 0, 0)
        for b in range(nblocks):
            n1, n0, nb = carry

            def live(c, b=b):
                n1, n0 = c
                m_copy(b, msems[b % 2]).wait()
                if b + 1 < nblocks:
                    m_copy(b + 1, msems[(b + 1) % 2]).start()
                r_copy(b).start()

                def step(i, c):
                    n1, n0 = c
                    base = b * BLK + i * (UNROLL * L)
                    for u in range(UNROLL):
                        n1, n0 = scan_chunk(base + u * L, n1, n0)
                    return n1, n0

                n1, n0 = lax.fori_loop(0, BLK // (UNROLL * L), step, (n1, n0))
                return n1, n0, b + 1

            carry = lax.cond(
                (n1 < K) | (n0 < K), live, lambda c: (c[0], c[1], nb), (n1, n0)
            )
        n1f, _, nb = carry
        n1e = jnp.minimum(n1f, K)

        for b in range(1, nblocks):
            when(nb == b, lambda b=b: m_copy(b, msems[b % 2]).wait())

        def drain_rule(b, _):
            when(b < nb, lambda: r_copy(b).wait())
            return 0

        lax.fori_loop(0, nblocks, drain_rule, 0)

        def combine(t, _):
            j = t * L + iota
            take1 = j < n1e
            src = jnp.where(take1, j, j - n1e + K)
            sel = plsc.load_gather(comb_v, [src])
            sel_v[pl.ds(t * L, L)] = sel
            rlo_v[pl.ds(t * L, L)] = plsc.load_gather(rule_v, [sel])
            mko_v[pl.ds(t * L, L)] = take1.astype(jnp.int32)
            return 0

        lax.fori_loop(0, K // L, combine, 0)
        pltpu.make_async_copy(mko_v, mask_out_hbm.at[row], osem).start()
        pltpu.make_async_copy(rlo_v, rule_out_hbm.at[row], osem).start()

        last1 = plsc.load_gather(
            comb_v, [jnp.broadcast_to(jnp.maximum(n1e - 1, 0), (L,))])
        last0 = plsc.load_gather(
            comb_v, [jnp.broadcast_to(
                jnp.clip(2 * K - n1e - 1, K, 2 * K - 1), (L,))])
        pe1 = jnp.where(n1e > 0, jnp.max(last1), 0)
        pe0 = jnp.where(n1e < K, jnp.max(last0), 0)
        pex = jnp.maximum(pe1, pe0) + 1
        nbb = (pex + BLK - 1) // BLK

        def p_copy(d, b):
            buf = rule_v if d % 2 == 0 else mask_v
            return pltpu.make_async_copy(
                body_hbm.bitcast(jnp.int32).at[row, d, pl.ds(b * BLK, BLK)],
                buf.at[pl.ds(b * BLK, BLK)], psems[d % 2])

        def fire_plane(d):
            def fb(b, _):
                when(b < nbb, lambda: p_copy(d, b).start())
                return 0

            lax.fori_loop(0, nblocks, fb, 0)

        def wait_plane(d):
            def wb(b, _):
                when(b < nbb, lambda: p_copy(d, b).wait())
                return 0

            lax.fori_loop(0, nblocks, wb, 0)

        fire_plane(0)
        for d in range(D):
            wait_plane(d)
            if d + 1 < D:
                fire_plane(d + 1)
            if d >= 2:
                pltpu.make_async_copy(
                    obts[d % 2], body_out_hbm.at[row, d - 2], wsems[d % 2]
                ).wait()
            obt = obts[d % 2]
            pbuf = rule_v if d % 2 == 0 else mask_v

            def extract(t, _, obt=obt, pbuf=pbuf):
                nvec = sel_v[pl.ds(t * L, L)]
                vals = plsc.load_gather(pbuf, [nvec])
                obt[pl.ds(t * L, L)] = plsc.bitcast(vals, jnp.float32)
                return 0

            lax.fori_loop(0, K // L, extract, 0)
            pltpu.make_async_copy(
                obt, body_out_hbm.at[row, d], wsems[d % 2]).start()

        for d in (D - 2, D - 1):
            pltpu.make_async_copy(
                obts[d % 2], body_out_hbm.at[row, d], wsems[d % 2]).wait()
        pltpu.make_async_copy(mko_v, mask_out_hbm.at[row], osem).wait()
        pltpu.make_async_copy(rlo_v, rule_out_hbm.at[row], osem).wait()

    def row_iter(r, _):
        do_row(wid * rows_per_worker + r)
        return 0

    lax.fori_loop(0, rows_per_worker, row_iter, 0)


@jax.jit
def kernel(body, mask, rule_idx):
    B, N, D = body.shape
    rows_per_worker = B // NW
    rdt = rule_idx.dtype
    mask_i = mask.astype(jnp.int32)
    rule_i = rule_idx.astype(jnp.int32)
    body_t = body.transpose(0, 2, 1)

    mesh = plsc.VectorSubcoreMesh(
        core_axis_name="c", subcore_axis_name="s", num_cores=2, num_subcores=16
    )
    body_o, mask_o, rule_o = pl.kernel(
        functools.partial(_tec_body, N, D, rows_per_worker),
        out_type=(
            jax.ShapeDtypeStruct((B, D, K), jnp.float32),
            jax.ShapeDtypeStruct((B, K), jnp.int32),
            jax.ShapeDtypeStruct((B, K), jnp.int32),
        ),
        mesh=mesh,
        compiler_params=pltpu.CompilerParams(
            needs_layout_passes=False, use_tc_tiling_on_sc=True
        ),
        scratch_types=[
            pltpu.VMEM((N,), jnp.int32),
            pltpu.VMEM((N,), jnp.int32),
            pltpu.VMEM((2 * K,), jnp.int32),
            pltpu.VMEM((K,), jnp.int32),
            pltpu.VMEM((K,), jnp.int32),
            pltpu.VMEM((K,), jnp.int32),
            pltpu.VMEM((K,), jnp.float32),
            pltpu.VMEM((K,), jnp.float32),
            pltpu.SemaphoreType.DMA,
            pltpu.SemaphoreType.DMA,
            pltpu.SemaphoreType.DMA,
            pltpu.SemaphoreType.DMA,
            pltpu.SemaphoreType.DMA,
            pltpu.SemaphoreType.DMA,
            pltpu.SemaphoreType.DMA,
            pltpu.SemaphoreType.DMA,
        ],
    )(mask_i, rule_i, body_t)
    return body_o.transpose(0, 2, 1), mask_o.astype(jnp.bool_), rule_o.astype(rdt)


# speedup vs baseline: 2.1332x; 1.0221x over previous
import functools

import jax
import jax.numpy as jnp
from jax import lax
from jax.experimental import pallas as pl
from jax.experimental.pallas import tpu as pltpu
from jax.experimental.pallas import tpu_sc as plsc

K = 2048
L = 16
NW = 32
BLK = 8192
UNROLL = 4


def _tec_body(N, D, rows_per_worker,
              mask_hbm, rule_hbm, body_hbm,
              body_out_hbm, mask_out_hbm, rule_out_hbm,
              mask_v, rule_v, comb_v, sel_v, mko_v, rlo_v,
              obt0_v, obt1_v,
              msem0, msem1, rsem, psem0, psem1, wsem0, wsem1, osem):
    nblocks = N // BLK
    wid = lax.axis_index("s") * 2 + lax.axis_index("c")
    iota = lax.iota(jnp.int32, L)
    msems = (msem0, msem1)
    psems = (psem0, psem1)
    wsems = (wsem0, wsem1)
    obts = (obt0_v, obt1_v)

    def when(pred, fn):
        lax.cond(pred, fn, lambda: None)

    def do_row(row):
        def m_copy(b, sem):
            return pltpu.make_async_copy(
                mask_hbm.at[row, pl.ds(b * BLK, BLK)],
                mask_v.at[pl.ds(b * BLK, BLK)], sem)

        def r_copy(b):
            return pltpu.make_async_copy(
                rule_hbm.at[row, pl.ds(b * BLK, BLK)],
                rule_v.at[pl.ds(b * BLK, BLK)], rsem)

        m_copy(0, msems[0]).start()

        def scan_chunk(base_elem, n1v, n0v):
            m = mask_v[pl.ds(base_elem, L)]
            mb = m > 0
            incl = plsc.cumsum(m)
            excl = incl - m
            s = plsc.all_reduce_population_count(mb)
            g = base_elem + iota
            pos1 = n1v + excl
            pos0 = n0v + (iota - excl)
            m1 = mb & (pos1 < K)
            m0 = (~mb) & (pos0 < K)
            plsc.store_scatter(comb_v, [pos1], g, mask=m1)
            plsc.store_scatter(comb_v, [pos0 + K], g, mask=m0)
            return n1v + s, n0v + (L - s)

        carry = (0, 0, 0)
        for b in range(nblocks):
            n1, n0, nb = carry

            def live(c, b=b):
                n1, n0 = c
                m_copy(b, msems[b % 2]).wait()
                if b + 1 < nblocks:
                    m_copy(b + 1, msems[(b + 1) % 2]).start()
                r_copy(b).start()

                def step(i, c):
                    n1v, n0v = c
                    base = b * BLK + i * (UNROLL * L)
                    for u in range(UNROLL):
                        n1v, n0v = scan_chunk(base + u * L, n1v, n0v)
                    return n1v, n0v

                n1v, n0v = lax.fori_loop(
                    0, BLK // (UNROLL * L), step,
                    (jnp.full((L,), n1, jnp.int32),
                     jnp.full((L,), n0, jnp.int32)),
                )
                return jnp.max(n1v), jnp.max(n0v), b + 1

            carry = lax.cond(
                (n1 < K) | (n0 < K), live, lambda c: (c[0], c[1], nb), (n1, n0)
            )
        n1f, _, nb = carry
        n1e = jnp.minimum(n1f, K)

        for b in range(1, nblocks):
            when(nb == b, lambda b=b: m_copy(b, msems[b % 2]).wait())

        def drain_rule(b, _):
            when(b < nb, lambda: r_copy(b).wait())
            return 0

        lax.fori_loop(0, nblocks, drain_rule, 0)

        def combine(t, _):
            j = t * L + iota
            take1 = j < n1e
            src = jnp.where(take1, j, j - n1e + K)
            sel = plsc.load_gather(comb_v, [src])
            sel_v[pl.ds(t * L, L)] = sel
            rlo_v[pl.ds(t * L, L)] = plsc.load_gather(rule_v, [sel])
            mko_v[pl.ds(t * L, L)] = take1.astype(jnp.int32)
            return 0

        lax.fori_loop(0, K // L, combine, 0)
        pltpu.make_async_copy(mko_v, mask_out_hbm.at[row], osem).start()
        pltpu.make_async_copy(rlo_v, rule_out_hbm.at[row], osem).start()

        last1 = plsc.load_gather(
            comb_v, [jnp.broadcast_to(jnp.maximum(n1e - 1, 0), (L,))])
        last0 = plsc.load_gather(
            comb_v, [jnp.broadcast_to(
                jnp.clip(2 * K - n1e - 1, K, 2 * K - 1), (L,))])
        pe1 = jnp.where(n1e > 0, jnp.max(last1), 0)
        pe0 = jnp.where(n1e < K, jnp.max(last0), 0)
        pex = jnp.maximum(pe1, pe0) + 1
        nbb = (pex + BLK - 1) // BLK

        def p_copy(d, b):
            buf = rule_v if d % 2 == 0 else mask_v
            return pltpu.make_async_copy(
                body_hbm.bitcast(jnp.int32).at[row, d, pl.ds(b * BLK, BLK)],
                buf.at[pl.ds(b * BLK, BLK)], psems[d % 2])

        def fire_plane(d):
            def fb(b, _):
                when(b < nbb, lambda: p_copy(d, b).start())
                return 0

            lax.fori_loop(0, nblocks, fb, 0)

        def wait_plane(d):
            def wb(b, _):
                when(b < nbb, lambda: p_copy(d, b).wait())
                return 0

            lax.fori_loop(0, nblocks, wb, 0)

        fire_plane(0)
        for d in range(D):
            wait_plane(d)
            if d + 1 < D:
                fire_plane(d + 1)
            if d >= 2:
                pltpu.make_async_copy(
                    obts[d % 2], body_out_hbm.at[row, d - 2], wsems[d % 2]
                ).wait()
            obt = obts[d % 2]
            pbuf = rule_v if d % 2 == 0 else mask_v

            def extract(t, _, obt=obt, pbuf=pbuf):
                for u in range(UNROLL):
                    off = (t * UNROLL + u) * L
                    nvec = sel_v[pl.ds(off, L)]
                    vals = plsc.load_gather(pbuf, [nvec])
                    obt[pl.ds(off, L)] = plsc.bitcast(vals, jnp.float32)
                return 0

            lax.fori_loop(0, K // (UNROLL * L), extract, 0)
            pltpu.make_async_copy(
                obt, body_out_hbm.at[row, d], wsems[d % 2]).start()

        for d in (D - 2, D - 1):
            pltpu.make_async_copy(
                obts[d % 2], body_out_hbm.at[row, d], wsems[d % 2]).wait()
        pltpu.make_async_copy(mko_v, mask_out_hbm.at[row], osem).wait()
        pltpu.make_async_copy(rlo_v, rule_out_hbm.at[row], osem).wait()

    def row_iter(r, _):
        do_row(wid * rows_per_worker + r)
        return 0

    lax.fori_loop(0, rows_per_worker, row_iter, 0)


@jax.jit
def kernel(body, mask, rule_idx):
    B, N, D = body.shape
    rows_per_worker = B // NW
    rdt = rule_idx.dtype
    mask_i = mask.astype(jnp.int32)
    rule_i = rule_idx.astype(jnp.int32)
    body_t = body.transpose(0, 2, 1)

    mesh = plsc.VectorSubcoreMesh(
        core_axis_name="c", subcore_axis_name="s", num_cores=2, num_subcores=16
    )
    body_o, mask_o, rule_o = pl.kernel(
        functools.partial(_tec_body, N, D, rows_per_worker),
        out_type=(
            jax.ShapeDtypeStruct((B, D, K), jnp.float32),
            jax.ShapeDtypeStruct((B, K), jnp.int32),
            jax.ShapeDtypeStruct((B, K), jnp.int32),
        ),
        mesh=mesh,
        compiler_params=pltpu.CompilerParams(
            needs_layout_passes=False, use_tc_tiling_on_sc=True
        ),
        scratch_types=[
            pltpu.VMEM((N,), jnp.int32),
            pltpu.VMEM((N,), jnp.int32),
            pltpu.VMEM((2 * K,), jnp.int32),
            pltpu.VMEM((K,), jnp.int32),
            pltpu.VMEM((K,), jnp.int32),
            pltpu.VMEM((K,), jnp.int32),
            pltpu.VMEM((K,), jnp.float32),
            pltpu.VMEM((K,), jnp.float32),
            pltpu.SemaphoreType.DMA,
            pltpu.SemaphoreType.DMA,
            pltpu.SemaphoreType.DMA,
            pltpu.SemaphoreType.DMA,
            pltpu.SemaphoreType.DMA,
            pltpu.SemaphoreType.DMA,
            pltpu.SemaphoreType.DMA,
            pltpu.SemaphoreType.DMA,
        ],
    )(mask_i, rule_i, body_t)
    return body_o.transpose(0, 2, 1), mask_o.astype(jnp.bool_), rule_o.astype(rdt)


# speedup vs baseline: 2.4020x; 1.1260x over previous
import functools

import jax
import jax.numpy as jnp
from jax import lax
from jax.experimental import pallas as pl
from jax.experimental.pallas import tpu as pltpu
from jax.experimental.pallas import tpu_sc as plsc

K = 2048
L = 16
NW = 32
BLK = 8192
UNROLL = 4


def _tec_body(N, D, rows_per_worker,
              mask_hbm, rule_hbm, body_hbm,
              body_out_hbm, mask_out_hbm, rule_out_hbm,
              mask_v, rule_v, comb_v, sel_v, mko_v, rlo_v,
              obt0_v, obt1_v,
              msem0, msem1, rsem, psem0, psem1, wsem0, wsem1, osem):
    nblocks = N // BLK
    wid = lax.axis_index("s") * 2 + lax.axis_index("c")
    iota = lax.iota(jnp.int32, L)
    msems = (msem0, msem1)
    psems = (psem0, psem1)
    wsems = (wsem0, wsem1)
    obts = (obt0_v, obt1_v)

    def when(pred, fn):
        lax.cond(pred, fn, lambda: None)

    def do_row(row):
        def m_copy(b, sem):
            return pltpu.make_async_copy(
                mask_hbm.at[row, pl.ds(b * BLK, BLK)],
                mask_v.at[pl.ds(b * BLK, BLK)], sem)

        def r_copy(b):
            return pltpu.make_async_copy(
                rule_hbm.at[row, pl.ds(b * BLK, BLK)],
                rule_v.at[pl.ds(b * BLK, BLK)], rsem)

        m_copy(0, msems[0]).start()

        def scan_group(base_elem, n1v):
            ms = [mask_v[pl.ds(base_elem + u * L, L)] for u in range(UNROLL)]
            mbs = [m > 0 for m in ms]
            incls = [plsc.cumsum(m) for m in ms]
            ss = [plsc.all_reduce_population_count(mb) for mb in mbs]
            for u in range(UNROLL):
                excl = incls[u] - ms[u]
                pos1 = n1v + excl
                m1 = mbs[u] & (pos1 < K)
                g = base_elem + u * L + iota
                plsc.store_scatter(comb_v, [pos1], g, mask=m1)
                n1v = n1v + ss[u]
            return n1v

        carry = (0, 0)
        for b in range(nblocks):
            n1, nb = carry

            def live(c, b=b):
                n1 = c[0]
                m_copy(b, msems[b % 2]).wait()
                if b + 1 < nblocks:
                    m_copy(b + 1, msems[(b + 1) % 2]).start()
                r_copy(b).start()

                def step(i, n1v):
                    return scan_group(b * BLK + i * (UNROLL * L), n1v)

                n1v = lax.fori_loop(
                    0, BLK // (UNROLL * L), step,
                    jnp.full((L,), n1, jnp.int32),
                )
                return jnp.max(n1v), b + 1

            carry = lax.cond(n1 < K, live, lambda c: c, (n1, nb))
        n1f, nb = carry
        n1e = jnp.minimum(n1f, K)

        def zeros_pass():
            def zstep(i, n0v):
                m = mask_v[pl.ds(i * L, L)]
                mb = m > 0
                incl = plsc.cumsum(m)
                excl = incl - m
                pos0 = n0v + (iota - excl)
                m0 = (~mb) & (pos0 < K)
                plsc.store_scatter(comb_v, [pos0 + K], i * L + iota, mask=m0)
                return n0v + (L - plsc.all_reduce_population_count(mb))

            lax.fori_loop(0, N // L, zstep, jnp.zeros((L,), jnp.int32))

        lax.cond(n1f < K, zeros_pass, lambda: None)

        for b in range(1, nblocks):
            when(nb == b, lambda b=b: m_copy(b, msems[b % 2]).wait())

        def drain_rule(b, _):
            when(b < nb, lambda: r_copy(b).wait())
            return 0

        lax.fori_loop(0, nblocks, drain_rule, 0)

        def combine(t, _):
            j = t * L + iota
            take1 = j < n1e
            src = jnp.where(take1, j, j - n1e + K)
            sel = plsc.load_gather(comb_v, [src])
            sel_v[pl.ds(t * L, L)] = sel
            rlo_v[pl.ds(t * L, L)] = plsc.load_gather(rule_v, [sel])
            mko_v[pl.ds(t * L, L)] = take1.astype(jnp.int32)
            return 0

        lax.fori_loop(0, K // L, combine, 0)
        pltpu.make_async_copy(mko_v, mask_out_hbm.at[row], osem).start()
        pltpu.make_async_copy(rlo_v, rule_out_hbm.at[row], osem).start()

        last1 = plsc.load_gather(
            comb_v, [jnp.broadcast_to(jnp.maximum(n1e - 1, 0), (L,))])
        last0 = plsc.load_gather(
            comb_v, [jnp.broadcast_to(
                jnp.clip(2 * K - n1e - 1, K, 2 * K - 1), (L,))])
        pe1 = jnp.where(n1e > 0, jnp.max(last1), 0)
        pe0 = jnp.where(n1e < K, jnp.max(last0), 0)
        pex = jnp.maximum(pe1, pe0) + 1
        nbb = (pex + BLK - 1) // BLK

        def p_copy(d, b):
            buf = rule_v if d % 2 == 0 else mask_v
            return pltpu.make_async_copy(
                body_hbm.bitcast(jnp.int32).at[row, d, pl.ds(b * BLK, BLK)],
                buf.at[pl.ds(b * BLK, BLK)], psems[d % 2])

        def fire_plane(d):
            def fb(b, _):
                when(b < nbb, lambda: p_copy(d, b).start())
                return 0

            lax.fori_loop(0, nblocks, fb, 0)

        def wait_plane(d):
            def wb(b, _):
                when(b < nbb, lambda: p_copy(d, b).wait())
                return 0

            lax.fori_loop(0, nblocks, wb, 0)

        fire_plane(0)
        for d in range(D):
            wait_plane(d)
            if d + 1 < D:
                fire_plane(d + 1)
            if d >= 2:
                pltpu.make_async_copy(
                    obts[d % 2], body_out_hbm.at[row, d - 2], wsems[d % 2]
                ).wait()
            obt = obts[d % 2]
            pbuf = rule_v if d % 2 == 0 else mask_v

            XU = 8

            def extract(t, _, obt=obt, pbuf=pbuf):
                nvecs = [sel_v[pl.ds((t * XU + u) * L, L)] for u in range(XU)]
                vals = [plsc.load_gather(pbuf, [nv]) for nv in nvecs]
                for u in range(XU):
                    obt[pl.ds((t * XU + u) * L, L)] = plsc.bitcast(
                        vals[u], jnp.float32)
                return 0

            lax.fori_loop(0, K // (XU * L), extract, 0)
            pltpu.make_async_copy(
                obt, body_out_hbm.at[row, d], wsems[d % 2]).start()

        for d in (D - 2, D - 1):
            pltpu.make_async_copy(
                obts[d % 2], body_out_hbm.at[row, d], wsems[d % 2]).wait()
        pltpu.make_async_copy(mko_v, mask_out_hbm.at[row], osem).wait()
        pltpu.make_async_copy(rlo_v, rule_out_hbm.at[row], osem).wait()

    def row_iter(r, _):
        do_row(wid * rows_per_worker + r)
        return 0

    lax.fori_loop(0, rows_per_worker, row_iter, 0)


@jax.jit
def kernel(body, mask, rule_idx):
    B, N, D = body.shape
    rows_per_worker = B // NW
    rdt = rule_idx.dtype
    mask_i = mask.astype(jnp.int32)
    rule_i = rule_idx.astype(jnp.int32)
    body_t = body.transpose(0, 2, 1)

    mesh = plsc.VectorSubcoreMesh(
        core_axis_name="c", subcore_axis_name="s", num_cores=2, num_subcores=16
    )
    body_o, mask_o, rule_o = pl.kernel(
        functools.partial(_tec_body, N, D, rows_per_worker),
        out_type=(
            jax.ShapeDtypeStruct((B, D, K), jnp.float32),
            jax.ShapeDtypeStruct((B, K), jnp.int32),
            jax.ShapeDtypeStruct((B, K), jnp.int32),
        ),
        mesh=mesh,
        compiler_params=pltpu.CompilerParams(
            needs_layout_passes=False, use_tc_tiling_on_sc=True
        ),
        scratch_types=[
            pltpu.VMEM((N,), jnp.int32),
            pltpu.VMEM((N,), jnp.int32),
            pltpu.VMEM((2 * K,), jnp.int32),
            pltpu.VMEM((K,), jnp.int32),
            pltpu.VMEM((K,), jnp.int32),
            pltpu.VMEM((K,), jnp.int32),
            pltpu.VMEM((K,), jnp.float32),
            pltpu.VMEM((K,), jnp.float32),
            pltpu.SemaphoreType.DMA,
            pltpu.SemaphoreType.DMA,
            pltpu.SemaphoreType.DMA,
            pltpu.SemaphoreType.DMA,
            pltpu.SemaphoreType.DMA,
            pltpu.SemaphoreType.DMA,
            pltpu.SemaphoreType.DMA,
            pltpu.SemaphoreType.DMA,
        ],
    )(mask_i, rule_i, body_t)
    return body_o.transpose(0, 2, 1), mask_o.astype(jnp.bool_), rule_o.astype(rdt)


# speedup vs baseline: 2.4623x; 1.0251x over previous
import functools

import jax
import jax.numpy as jnp
from jax import lax
from jax.experimental import pallas as pl
from jax.experimental.pallas import tpu as pltpu
from jax.experimental.pallas import tpu_sc as plsc

K = 2048
L = 16
NW = 32
BLK = 8192
UNROLL = 4
PBLK = 2048


def _tec_body(N, D, rows_per_worker,
              mask_hbm, rule_hbm, body_hbm,
              body_out_hbm, mask_out_hbm, rule_out_hbm,
              mask_v, rule_v, comb_v, sel_v, mko_v, rlo_v,
              obt0_v, obt1_v,
              msem0, msem1, rsem, psem0, psem1, wsem0, wsem1, osem):
    nblocks = N // BLK
    wid = lax.axis_index("s") * 2 + lax.axis_index("c")
    iota = lax.iota(jnp.int32, L)
    msems = (msem0, msem1)
    psems = (psem0, psem1)
    wsems = (wsem0, wsem1)
    obts = (obt0_v, obt1_v)

    def when(pred, fn):
        lax.cond(pred, fn, lambda: None)

    def do_row(row):
        def m_copy(b, sem):
            return pltpu.make_async_copy(
                mask_hbm.at[row, pl.ds(b * BLK, BLK)],
                mask_v.at[pl.ds(b * BLK, BLK)], sem)

        def r_copy(b):
            return pltpu.make_async_copy(
                rule_hbm.at[row, pl.ds(b * BLK, BLK)],
                rule_v.at[pl.ds(b * BLK, BLK)], rsem)

        m_copy(0, msems[0]).start()

        def scan_group(base_elem, n1v):
            ms = [mask_v[pl.ds(base_elem + u * L, L)] for u in range(UNROLL)]
            mbs = [m > 0 for m in ms]
            incls = [plsc.cumsum(m) for m in ms]
            ss = [plsc.all_reduce_population_count(mb) for mb in mbs]
            for u in range(UNROLL):
                excl = incls[u] - ms[u]
                pos1 = n1v + excl
                m1 = mbs[u] & (pos1 < K)
                g = base_elem + u * L + iota
                plsc.store_scatter(comb_v, [pos1], g, mask=m1)
                n1v = n1v + ss[u]
            return n1v

        carry = (0, 0)
        for b in range(nblocks):
            n1, nb = carry

            def live(c, b=b):
                n1 = c[0]
                m_copy(b, msems[b % 2]).wait()
                if b + 1 < nblocks:
                    m_copy(b + 1, msems[(b + 1) % 2]).start()
                r_copy(b).start()

                def step(i, n1v):
                    return scan_group(b * BLK + i * (UNROLL * L), n1v)

                n1v = lax.fori_loop(
                    0, BLK // (UNROLL * L), step,
                    jnp.full((L,), n1, jnp.int32),
                )
                return jnp.max(n1v), b + 1

            carry = lax.cond(n1 < K, live, lambda c: c, (n1, nb))
        n1f, nb = carry
        n1e = jnp.minimum(n1f, K)

        def zeros_pass():
            def zstep(i, n0v):
                m = mask_v[pl.ds(i * L, L)]
                mb = m > 0
                incl = plsc.cumsum(m)
                excl = incl - m
                pos0 = n0v + (iota - excl)
                m0 = (~mb) & (pos0 < K)
                plsc.store_scatter(comb_v, [pos0 + K], i * L + iota, mask=m0)
                return n0v + (L - plsc.all_reduce_population_count(mb))

            lax.fori_loop(0, N // L, zstep, jnp.zeros((L,), jnp.int32))

        lax.cond(n1f < K, zeros_pass, lambda: None)

        for b in range(1, nblocks):
            when(nb == b, lambda b=b: m_copy(b, msems[b % 2]).wait())

        def drain_rule(b, _):
            when(b < nb, lambda: r_copy(b).wait())
            return 0

        lax.fori_loop(0, nblocks, drain_rule, 0)

        last1 = plsc.load_gather(
            comb_v, [jnp.broadcast_to(jnp.maximum(n1e - 1, 0), (L,))])
        last0 = plsc.load_gather(
            comb_v, [jnp.broadcast_to(
                jnp.clip(2 * K - n1e - 1, K, 2 * K - 1), (L,))])
        pe1 = jnp.where(n1e > 0, jnp.max(last1), 0)
        pe0 = jnp.where(n1e < K, jnp.max(last0), 0)
        pex = jnp.maximum(pe1, pe0) + 1
        nbb = (pex + PBLK - 1) // PBLK

        def p_copy(d, b):
            buf = mask_v if d % 2 == 0 else rule_v
            return pltpu.make_async_copy(
                body_hbm.bitcast(jnp.int32).at[row, d, pl.ds(b * PBLK, PBLK)],
                buf.at[pl.ds(b * PBLK, PBLK)], psems[d % 2])

        def fire_plane(d):
            def fb(b, _):
                when(b < nbb, lambda: p_copy(d, b).start())
                return 0

            lax.fori_loop(0, N // PBLK, fb, 0)

        def wait_plane(d):
            def wb(b, _):
                when(b < nbb, lambda: p_copy(d, b).wait())
                return 0

            lax.fori_loop(0, N // PBLK, wb, 0)

        fire_plane(0)

        def combine(t, _):
            j = t * L + iota
            take1 = j < n1e
            src = jnp.where(take1, j, j - n1e + K)
            sel = plsc.load_gather(comb_v, [src])
            sel_v[pl.ds(t * L, L)] = sel
            rlo_v[pl.ds(t * L, L)] = plsc.load_gather(rule_v, [sel])
            mko_v[pl.ds(t * L, L)] = take1.astype(jnp.int32)
            return 0

        lax.fori_loop(0, K // L, combine, 0)
        pltpu.make_async_copy(mko_v, mask_out_hbm.at[row], osem).start()
        pltpu.make_async_copy(rlo_v, rule_out_hbm.at[row], osem).start()

        for d in range(D):
            wait_plane(d)
            if d + 1 < D:
                fire_plane(d + 1)
            if d >= 2:
                pltpu.make_async_copy(
                    obts[d % 2], body_out_hbm.at[row, d - 2], wsems[d % 2]
                ).wait()
            obt = obts[d % 2]
            pbuf = mask_v if d % 2 == 0 else rule_v

            XU = 8

            def extract(t, _, obt=obt, pbuf=pbuf):
                nvecs = [sel_v[pl.ds((t * XU + u) * L, L)] for u in range(XU)]
                vals = [plsc.load_gather(pbuf, [nv]) for nv in nvecs]
                for u in range(XU):
                    obt[pl.ds((t * XU + u) * L, L)] = plsc.bitcast(
                        vals[u], jnp.float32)
                return 0

            lax.fori_loop(0, K // (XU * L), extract, 0)
            pltpu.make_async_copy(
                obt, body_out_hbm.at[row, d], wsems[d % 2]).start()

        for d in (D - 2, D - 1):
            pltpu.make_async_copy(
                obts[d % 2], body_out_hbm.at[row, d], wsems[d % 2]).wait()
        pltpu.make_async_copy(mko_v, mask_out_hbm.at[row], osem).wait()
        pltpu.make_async_copy(rlo_v, rule_out_hbm.at[row], osem).wait()

    def row_iter(r, _):
        do_row(wid * rows_per_worker + r)
        return 0

    lax.fori_loop(0, rows_per_worker, row_iter, 0)


@jax.jit
def kernel(body, mask, rule_idx):
    B, N, D = body.shape
    rows_per_worker = B // NW
    rdt = rule_idx.dtype
    mask_i = mask.astype(jnp.int32)
    rule_i = rule_idx.astype(jnp.int32)
    body_t = body.transpose(0, 2, 1)

    mesh = plsc.VectorSubcoreMesh(
        core_axis_name="c", subcore_axis_name="s", num_cores=2, num_subcores=16
    )
    body_o, mask_o, rule_o = pl.kernel(
        functools.partial(_tec_body, N, D, rows_per_worker),
        out_type=(
            jax.ShapeDtypeStruct((B, D, K), jnp.float32),
            jax.ShapeDtypeStruct((B, K), jnp.int32),
            jax.ShapeDtypeStruct((B, K), jnp.int32),
        ),
        mesh=mesh,
        compiler_params=pltpu.CompilerParams(
            needs_layout_passes=False, use_tc_tiling_on_sc=True
        ),
        scratch_types=[
            pltpu.VMEM((N,), jnp.int32),
            pltpu.VMEM((N,), jnp.int32),
            pltpu.VMEM((2 * K,), jnp.int32),
            pltpu.VMEM((K,), jnp.int32),
            pltpu.VMEM((K,), jnp.int32),
            pltpu.VMEM((K,), jnp.int32),
            pltpu.VMEM((K,), jnp.float32),
            pltpu.VMEM((K,), jnp.float32),
            pltpu.SemaphoreType.DMA,
            pltpu.SemaphoreType.DMA,
            pltpu.SemaphoreType.DMA,
            pltpu.SemaphoreType.DMA,
            pltpu.SemaphoreType.DMA,
            pltpu.SemaphoreType.DMA,
            pltpu.SemaphoreType.DMA,
            pltpu.SemaphoreType.DMA,
        ],
    )(mask_i, rule_i, body_t)
    return body_o.transpose(0, 2, 1), mask_o.astype(jnp.bool_), rule_o.astype(rdt)


# speedup vs baseline: 2.4823x; 1.0081x over previous
import functools

import jax
import jax.numpy as jnp
from jax import lax
from jax.experimental import pallas as pl
from jax.experimental.pallas import tpu as pltpu
from jax.experimental.pallas import tpu_sc as plsc

K = 2048
L = 16
NW = 32
BLK = 8192
UNROLL = 4
PBLK = 2048


def _tec_body(N, D, rows_per_worker,
              mask_hbm, rule_hbm, body_hbm,
              body_out_hbm, mask_out_hbm, rule_out_hbm,
              mask_v, rule_v, pf_v, comb_v, sel_v, mko_v, rlo_v,
              obt0_v, obt1_v,
              msem0, msem1, rsem, psem0, psem1, wsem0, wsem1, osem, pfsem):
    nblocks = N // BLK
    wid = lax.axis_index("s") * 2 + lax.axis_index("c")
    iota = lax.iota(jnp.int32, L)
    msems = (msem0, msem1)
    psems = (psem0, psem1)
    wsems = (wsem0, wsem1)
    obts = (obt0_v, obt1_v)

    def when(pred, fn):
        lax.cond(pred, fn, lambda: None)

    def m0_copy(row):
        return pltpu.make_async_copy(
            mask_hbm.at[row, pl.ds(0, BLK)], pf_v, pfsem)

    def do_row(r, row):
        def m_copy(b, sem):
            return pltpu.make_async_copy(
                mask_hbm.at[row, pl.ds(b * BLK, BLK)],
                mask_v.at[pl.ds(b * BLK, BLK)], sem)

        def r_copy(b):
            return pltpu.make_async_copy(
                rule_hbm.at[row, pl.ds(b * BLK, BLK)],
                rule_v.at[pl.ds(b * BLK, BLK)], rsem)

        def scan_group(buf, local_base, global_base, n1v):
            ms = [buf[pl.ds(local_base + u * L, L)] for u in range(UNROLL)]
            mbs = [m > 0 for m in ms]
            incls = [plsc.cumsum(m) for m in ms]
            ss = [plsc.all_reduce_population_count(mb) for mb in mbs]
            for u in range(UNROLL):
                excl = incls[u] - ms[u]
                pos1 = n1v + excl
                m1 = mbs[u] & (pos1 < K)
                g = global_base + u * L + iota
                plsc.store_scatter(comb_v, [pos1], g, mask=m1)
                n1v = n1v + ss[u]
            return n1v

        carry = (0, 0)
        for b in range(nblocks):
            n1, nb = carry

            def live(c, b=b):
                n1 = c[0]
                if b == 0:
                    m0_copy(row).wait()
                else:
                    m_copy(b, msems[b % 2]).wait()
                if b + 1 < nblocks:
                    m_copy(b + 1, msems[(b + 1) % 2]).start()
                r_copy(b).start()

                def step(i, n1v):
                    lb = i * (UNROLL * L)
                    gb = b * BLK + lb
                    if b == 0:
                        return scan_group(pf_v, lb, gb, n1v)
                    return scan_group(mask_v, gb, gb, n1v)

                n1v = lax.fori_loop(
                    0, BLK // (UNROLL * L), step,
                    jnp.full((L,), n1, jnp.int32),
                )
                return jnp.max(n1v), b + 1

            carry = lax.cond(n1 < K, live, lambda c: c, (n1, nb))
        n1f, nb = carry
        n1e = jnp.minimum(n1f, K)

        def zeros_pass():
            def zstep0(i, n0v):
                m = pf_v[pl.ds(i * L, L)]
                mb = m > 0
                incl = plsc.cumsum(m)
                excl = incl - m
                pos0 = n0v + (iota - excl)
                m0 = (~mb) & (pos0 < K)
                plsc.store_scatter(comb_v, [pos0 + K], i * L + iota, mask=m0)
                return n0v + (L - plsc.all_reduce_population_count(mb))

            def zstep(i, n0v):
                m = mask_v[pl.ds(i * L, L)]
                mb = m > 0
                incl = plsc.cumsum(m)
                excl = incl - m
                pos0 = n0v + (iota - excl)
                m0 = (~mb) & (pos0 < K)
                plsc.store_scatter(comb_v, [pos0 + K], i * L + iota, mask=m0)
                return n0v + (L - plsc.all_reduce_population_count(mb))

            n0v = lax.fori_loop(
                0, BLK // L, zstep0, jnp.zeros((L,), jnp.int32))
            lax.fori_loop(BLK // L, N // L, zstep, n0v)

        lax.cond(n1f < K, zeros_pass, lambda: None)

        if rows_per_worker > 1:
            when(r + 1 < rows_per_worker, lambda: m0_copy(row + 1).start())

        for b in range(1, nblocks):
            when(nb == b, lambda b=b: m_copy(b, msems[b % 2]).wait())

        def drain_rule(b, _):
            when(b < nb, lambda: r_copy(b).wait())
            return 0

        lax.fori_loop(0, nblocks, drain_rule, 0)

        last1 = plsc.load_gather(
            comb_v, [jnp.broadcast_to(jnp.maximum(n1e - 1, 0), (L,))])
        last0 = plsc.load_gather(
            comb_v, [jnp.broadcast_to(
                jnp.clip(2 * K - n1e - 1, K, 2 * K - 1), (L,))])
        pe1 = jnp.where(n1e > 0, jnp.max(last1), 0)
        pe0 = jnp.where(n1e < K, jnp.max(last0), 0)
        pex = jnp.maximum(pe1, pe0) + 1
        nbb = (pex + PBLK - 1) // PBLK

        def p_copy(d, b):
            buf = mask_v if d % 2 == 0 else rule_v
            return pltpu.make_async_copy(
                body_hbm.bitcast(jnp.int32).at[row, d, pl.ds(b * PBLK, PBLK)],
                buf.at[pl.ds(b * PBLK, PBLK)], psems[d % 2])

        def fire_plane(d):
            def fb(b, _):
                when(b < nbb, lambda: p_copy(d, b).start())
                return 0

            lax.fori_loop(0, N // PBLK, fb, 0)

        def wait_plane(d):
            def wb(b, _):
                when(b < nbb, lambda: p_copy(d, b).wait())
                return 0

            lax.fori_loop(0, N // PBLK, wb, 0)

        fire_plane(0)

        def combine(t, _):
            j = t * L + iota
            take1 = j < n1e
            src = jnp.where(take1, j, j - n1e + K)
            sel = plsc.load_gather(comb_v, [src])
            sel_v[pl.ds(t * L, L)] = sel
            rlo_v[pl.ds(t * L, L)] = plsc.load_gather(rule_v, [sel])
            mko_v[pl.ds(t * L, L)] = take1.astype(jnp.int32)
            return 0

        lax.fori_loop(0, K // L, combine, 0)
        pltpu.make_async_copy(mko_v, mask_out_hbm.at[row], osem).start()
        pltpu.make_async_copy(rlo_v, rule_out_hbm.at[row], osem).start()

        for d in range(D):
            wait_plane(d)
            if d + 1 < D:
                fire_plane(d + 1)
            if d >= 2:
                pltpu.make_async_copy(
                    obts[d % 2], body_out_hbm.at[row, d - 2], wsems[d % 2]
                ).wait()
            obt = obts[d % 2]
            pbuf = mask_v if d % 2 == 0 else rule_v

            XU = 8

            def extract(t, _, obt=obt, pbuf=pbuf):
                nvecs = [sel_v[pl.ds((t * XU + u) * L, L)] for u in range(XU)]
                vals = [plsc.load_gather(pbuf, [nv]) for nv in nvecs]
                for u in range(XU):
                    obt[pl.ds((t * XU + u) * L, L)] = plsc.bitcast(
                        vals[u], jnp.float32)
                return 0

            lax.fori_loop(0, K // (XU * L), extract, 0)
            pltpu.make_async_copy(
                obt, body_out_hbm.at[row, d], wsems[d % 2]).start()

        for d in (D - 2, D - 1):
            pltpu.make_async_copy(
                obts[d % 2], body_out_hbm.at[row, d], wsems[d % 2]).wait()
        pltpu.make_async_copy(mko_v, mask_out_hbm.at[row], osem).wait()
        pltpu.make_async_copy(rlo_v, rule_out_hbm.at[row], osem).wait()

    m0_copy(wid * rows_per_worker).start()

    def row_iter(r, _):
        do_row(r, wid * rows_per_worker + r)
        return 0

    lax.fori_loop(0, rows_per_worker, row_iter, 0)


@jax.jit
def kernel(body, mask, rule_idx):
    B, N, D = body.shape
    rows_per_worker = B // NW
    rdt = rule_idx.dtype
    mask_i = mask.astype(jnp.int32)
    rule_i = rule_idx.astype(jnp.int32)
    body_t = body.transpose(0, 2, 1)

    mesh = plsc.VectorSubcoreMesh(
        core_axis_name="c", subcore_axis_name="s", num_cores=2, num_subcores=16
    )
    body_o, mask_o, rule_o = pl.kernel(
        functools.partial(_tec_body, N, D, rows_per_worker),
        out_type=(
            jax.ShapeDtypeStruct((B, D, K), jnp.float32),
            jax.ShapeDtypeStruct((B, K), jnp.int32),
            jax.ShapeDtypeStruct((B, K), jnp.int32),
        ),
        mesh=mesh,
        compiler_params=pltpu.CompilerParams(
            needs_layout_passes=False, use_tc_tiling_on_sc=True
        ),
        scratch_types=[
            pltpu.VMEM((N,), jnp.int32),
            pltpu.VMEM((N,), jnp.int32),
            pltpu.VMEM((BLK,), jnp.int32),
            pltpu.VMEM((2 * K,), jnp.int32),
            pltpu.VMEM((K,), jnp.int32),
            pltpu.VMEM((K,), jnp.int32),
            pltpu.VMEM((K,), jnp.int32),
            pltpu.VMEM((K,), jnp.float32),
            pltpu.VMEM((K,), jnp.float32),
            pltpu.SemaphoreType.DMA,
            pltpu.SemaphoreType.DMA,
            pltpu.SemaphoreType.DMA,
            pltpu.SemaphoreType.DMA,
            pltpu.SemaphoreType.DMA,
            pltpu.SemaphoreType.DMA,
            pltpu.SemaphoreType.DMA,
            pltpu.SemaphoreType.DMA,
            pltpu.SemaphoreType.DMA,
        ],
    )(mask_i, rule_i, body_t)
    return body_o.transpose(0, 2, 1), mask_o.astype(jnp.bool_), rule_o.astype(rdt)
